# Initial kernel scaffold; baseline (speedup 1.0000x reference)
import jax
import jax.numpy as jnp
from jax.experimental import pallas as pl


def kernel(states, action_categories, play_card_ids, attacking_card_ids, attacked_card_ids, evolving_card_ids, emb1, emb2, emb3, W1, b1, W2, b2):
    raise NotImplementedError("write your pallas kernel here")


# speedup vs baseline: 50.7167x; 50.7167x over previous
import functools

import jax
import jax.numpy as jnp
from jax import lax
from jax.experimental import pallas as pl
from jax.experimental.pallas import tpu as pltpu
from jax.experimental.pallas import tpu_sc as plsc

_B = 16384
_L = 20
_MID = 128
_NC = 2
_NS = 16
_NW = _NC * _NS
_BPW = _B // _NW
_GRP = _BPW // 16

_T1P, _T2P, _T3P = 16, 3008, 1024


def _prep_body(states_ref, emb1_ref, emb2_ref, emb3_ref, w1_ref, b1_ref,
               w2_ref, b2_ref, base_ref, t1_ref, t2_ref, t3a_ref, t3d_ref,
               t3e_ref):
    hi = lax.Precision.HIGHEST
    w2 = w2_ref[...]
    v = lax.dot_general(w2, w1_ref[...], (((1,), (0,)), ((), ())),
                        precision=hi)
    c = jnp.sum(b1_ref[...] * w2) + b2_ref[0, 0]

    def proj(emb, vk):
        return lax.dot_general(emb, vk, (((1,), (1,)), ((), ())),
                               precision=hi)

    base_ref[...] = proj(states_ref[...], v[:, 0:128])
    base_ref[...] = base_ref[...] + c

    @pl.when(pl.program_id(0) == 0)
    def _():
        z = jnp.zeros
        t1_ref[...] = jnp.concatenate(
            [proj(emb1_ref[...], v[:, 128:256]), z((11, 1), jnp.float32)], 0)
        t2_ref[...] = jnp.concatenate(
            [proj(emb2_ref[...], v[:, 256:384]), z((8, 1), jnp.float32)], 0)
        t3a_ref[...] = jnp.concatenate(
            [proj(emb3_ref[...], v[:, 384:512]), z((24, 1), jnp.float32)], 0)
        t3d_ref[...] = jnp.concatenate(
            [proj(emb3_ref[...], v[:, 512:640]), z((24, 1), jnp.float32)], 0)
        t3e_ref[...] = jnp.concatenate(
            [proj(emb3_ref[...], v[:, 640:768]), z((24, 1), jnp.float32)], 0)


_prep = pl.pallas_call(
    _prep_body,
    grid=(16,),
    in_specs=[
        pl.BlockSpec((1024, 128), lambda i: (i, 0)),
        pl.BlockSpec((5, 128), lambda i: (0, 0)),
        pl.BlockSpec((3000, 128), lambda i: (0, 0)),
        pl.BlockSpec((1000, 128), lambda i: (0, 0)),
        pl.BlockSpec((128, 768), lambda i: (0, 0)),
        pl.BlockSpec((1, 128), lambda i: (0, 0)),
        pl.BlockSpec((1, 128), lambda i: (0, 0)),
        pl.BlockSpec((1, 1), lambda i: (0, 0)),
    ],
    out_specs=[
        pl.BlockSpec((1024, 1), lambda i: (i, 0)),
        pl.BlockSpec((_T1P, 1), lambda i: (0, 0)),
        pl.BlockSpec((_T2P, 1), lambda i: (0, 0)),
        pl.BlockSpec((_T3P, 1), lambda i: (0, 0)),
        pl.BlockSpec((_T3P, 1), lambda i: (0, 0)),
        pl.BlockSpec((_T3P, 1), lambda i: (0, 0)),
    ],
    out_shape=[
        jax.ShapeDtypeStruct((_B, 1), jnp.float32),
        jax.ShapeDtypeStruct((_T1P, 1), jnp.float32),
        jax.ShapeDtypeStruct((_T2P, 1), jnp.float32),
        jax.ShapeDtypeStruct((_T3P, 1), jnp.float32),
        jax.ShapeDtypeStruct((_T3P, 1), jnp.float32),
        jax.ShapeDtypeStruct((_T3P, 1), jnp.float32),
    ],
)


def _make_sc_gather():
    mesh = plsc.VectorSubcoreMesh(core_axis_name="c", subcore_axis_name="s")

    @functools.partial(
        pl.kernel,
        mesh=mesh,
        out_type=jax.ShapeDtypeStruct((_B,), jnp.float32),
        compiler_params=pltpu.CompilerParams(needs_layout_passes=False),
        scratch_types=[
            pltpu.VMEM((_BPW * _L,), jnp.int32),
            pltpu.VMEM((_BPW * _L,), jnp.int32),
            pltpu.VMEM((_BPW * _L,), jnp.int32),
            pltpu.VMEM((_BPW * _L,), jnp.int32),
            pltpu.VMEM((_BPW * _L,), jnp.int32),
            pltpu.VMEM((_T1P,), jnp.float32),
            pltpu.VMEM((_T2P,), jnp.float32),
            pltpu.VMEM((_T3P,), jnp.float32),
            pltpu.VMEM((_T3P,), jnp.float32),
            pltpu.VMEM((_T3P,), jnp.float32),
            pltpu.VMEM((_BPW,), jnp.float32),
            pltpu.VMEM((_BPW,), jnp.float32),
        ],
    )
    def sc_k(ac_hbm, play_hbm, atk_hbm, dfd_hbm, evo_hbm, base_hbm,
             t1_hbm, t2_hbm, t3a_hbm, t3d_hbm, t3e_hbm, out_hbm,
             ac_v, play_v, atk_v, dfd_v, evo_v,
             t1_v, t2_v, t3a_v, t3d_v, t3e_v, base_v, out_v):
        wid = lax.axis_index("s") * _NC + lax.axis_index("c")
        b0 = wid * _BPW
        i0 = b0 * _L
        pltpu.sync_copy(ac_hbm.at[pl.ds(i0, _BPW * _L)], ac_v)
        pltpu.sync_copy(play_hbm.at[pl.ds(i0, _BPW * _L)], play_v)
        pltpu.sync_copy(atk_hbm.at[pl.ds(i0, _BPW * _L)], atk_v)
        pltpu.sync_copy(dfd_hbm.at[pl.ds(i0, _BPW * _L)], dfd_v)
        pltpu.sync_copy(evo_hbm.at[pl.ds(i0, _BPW * _L)], evo_v)
        pltpu.sync_copy(t1_hbm, t1_v)
        pltpu.sync_copy(t2_hbm, t2_v)
        pltpu.sync_copy(t3a_hbm, t3a_v)
        pltpu.sync_copy(t3d_hbm, t3d_v)
        pltpu.sync_copy(t3e_hbm, t3e_v)
        pltpu.sync_copy(base_hbm.at[pl.ds(b0, _BPW)], base_v)

        j20 = lax.iota(jnp.int32, 16) * _L

        def group(g, carry):
            acc = base_v[pl.ds(g * 16, 16)]
            a0 = g * (16 * _L)
            for l in range(_L):
                addr = j20 + (a0 + l)
                acc = acc + plsc.load_gather(
                    t1_v, [plsc.load_gather(ac_v, [addr])])
                acc = acc + plsc.load_gather(
                    t2_v, [plsc.load_gather(play_v, [addr])])
                acc = acc + plsc.load_gather(
                    t3a_v, [plsc.load_gather(atk_v, [addr])])
                acc = acc + plsc.load_gather(
                    t3d_v, [plsc.load_gather(dfd_v, [addr])])
                acc = acc + plsc.load_gather(
                    t3e_v, [plsc.load_gather(evo_v, [addr])])
            out_v[pl.ds(g * 16, 16)] = acc
            return carry

        lax.fori_loop(0, _GRP, group, 0)
        pltpu.sync_copy(out_v, out_hbm.at[pl.ds(b0, _BPW)])

    return sc_k


_sc_gather = _make_sc_gather()


def kernel(states, action_categories, play_card_ids, attacking_card_ids,
           attacked_card_ids, evolving_card_ids, emb1, emb2, emb3,
           W1, b1, W2, b2):
    base, t1, t2, t3a, t3d, t3e = _prep(
        states, emb1, emb2, emb3, W1, b1.reshape(1, _MID), W2,
        b2.reshape(1, 1))
    i32 = jnp.int32
    out = _sc_gather(
        action_categories.astype(i32).reshape(-1),
        play_card_ids.astype(i32).reshape(-1),
        attacking_card_ids.astype(i32).reshape(-1),
        attacked_card_ids.astype(i32).reshape(-1),
        evolving_card_ids.astype(i32).reshape(-1),
        base.reshape(-1),
        t1.reshape(-1), t2.reshape(-1),
        t3a.reshape(-1), t3d.reshape(-1), t3e.reshape(-1))
    return out.reshape(_B, 1)
